# baseline (speedup 1.0000x reference)
import functools

import jax
import jax.numpy as jnp
from jax import lax
from jax.experimental import pallas as pl
from jax.experimental.pallas import tpu as pltpu
from jax.experimental.pallas import tpu_sc as plsc

F32 = jnp.float32

N = 10000
E = 320000
D = 128
H = 128
DID = 32
DDYN = 16
B = 1024
L = 50
STEPS = 3

NC = 2
NS = 16
NW = NC * NS

NP = 10112
EK = 125
ECPT = E // EK // NW
IG = 8
NG = ECPT // IG
ROWS_PT = NP // NS

LB = B * L
GK = 80
GCPT = LB // GK // NW

RB = 2000
TSTEP = 10


def _sc_scatter_add(h, edges4d, zeros_tile):
  mesh = plsc.VectorSubcoreMesh(core_axis_name="c", subcore_axis_name="s",
                                num_cores=NC, num_subcores=NS)

  @functools.partial(
      pl.kernel,
      out_type=[jax.ShapeDtypeStruct((NP, H), F32),
                jax.ShapeDtypeStruct((NP, H), F32)],
      mesh=mesh,
      scratch_types=[
          pltpu.VMEM((2, IG, EK), jnp.int32),
          pltpu.VMEM((2, IG, EK), jnp.int32),
          pltpu.VMEM((EK, H), F32),
          pltpu.VMEM((EK, H), F32),
          pltpu.VMEM_SHARED((NP, H), F32),
          pltpu.SemaphoreType.DMA,
          pltpu.SemaphoreType.DMA,
          pltpu.SemaphoreType.DMA,
          pltpu.SemaphoreType.DMA,
      ],
  )
  def scatter_kernel(h_hbm, edges_hbm, z_hbm, out0, out1,
                     sidx, didx, rows0, rows1, acc_sh, gs0, gs1, is0, is1):
    cid = lax.axis_index("c")
    sid = lax.axis_index("s")
    wid = cid * NS + sid
    my_rows = pl.ds(sid * ROWS_PT, ROWS_PT)
    src_hbm = edges_hbm.at[0]
    dst_hbm = edges_hbm.at[1]
    pltpu.sync_copy(z_hbm, acc_sh.at[my_rows])
    pltpu.sync_copy(src_hbm.at[wid, pl.ds(0, IG)], sidx.at[0])
    pltpu.sync_copy(dst_hbm.at[wid, pl.ds(0, IG)], didx.at[0])
    plsc.subcore_barrier()

    bufs = (rows0, rows1)
    gsems = (gs0, gs1)
    SPLITS = ((0, 64), (64, EK - 64))

    def fire(gb_, k_, buf, sem):
      for (o, n) in SPLITS:
        pltpu.async_copy(h_hbm.at[sidx.at[gb_, k_, pl.ds(o, n)]],
                         buf.at[pl.ds(o, n)], sem)

    def drain(gb_, k_, buf, sem):
      for (o, n) in SPLITS:
        pltpu.make_async_copy(h_hbm.at[sidx.at[gb_, k_, pl.ds(o, n)]],
                              buf.at[pl.ds(o, n)], sem).wait()

    fire(0, 0, rows0, gs0)

    def group(g, carry):
      gb = g % 2
      ngb = (g + 1) % 2
      nxt = pl.ds((g + 1) * IG, IG)

      @pl.when(g + 1 < NG)
      def _():
        pltpu.async_copy(src_hbm.at[wid, nxt], sidx.at[ngb], is0)
        pltpu.async_copy(dst_hbm.at[wid, nxt], didx.at[ngb], is1)

      for k in range(IG):
        if k < IG - 1:
          fire(gb, k + 1, bufs[(k + 1) % 2], gsems[(k + 1) % 2])
        else:

          @pl.when(g + 1 < NG)
          def _():
            pltpu.make_async_copy(src_hbm.at[wid, nxt], sidx.at[ngb],
                                  is0).wait()
            pltpu.make_async_copy(dst_hbm.at[wid, nxt], didx.at[ngb],
                                  is1).wait()
            fire(ngb, 0, bufs[0], gsems[0])

        drain(gb, k, bufs[k % 2], gsems[k % 2])
        pltpu.sync_copy(bufs[k % 2], acc_sh.at[didx.at[gb, k]], add=True)
      return carry

    lax.fori_loop(0, NG, group, 0)
    plsc.subcore_barrier()

    @pl.when(cid == 0)
    def _():
      pltpu.sync_copy(acc_sh.at[my_rows], out0.at[my_rows])

    @pl.when(cid == 1)
    def _():
      pltpu.sync_copy(acc_sh.at[my_rows], out1.at[my_rows])

  return scatter_kernel(h, edges4d, zeros_tile)


def _sc_traj_gather(hcat, traj3d):
  mesh = plsc.VectorSubcoreMesh(core_axis_name="c", subcore_axis_name="s",
                                num_cores=NC, num_subcores=NS)

  @functools.partial(
      pl.kernel,
      out_type=jax.ShapeDtypeStruct((LB, 2 * H), F32),
      mesh=mesh,
      scratch_types=[
          pltpu.VMEM((GCPT, GK), jnp.int32),
          pltpu.VMEM((GK, 2 * H), F32),
          pltpu.VMEM((GK, 2 * H), F32),
          pltpu.SemaphoreType.DMA,
          pltpu.SemaphoreType.DMA,
          pltpu.SemaphoreType.DMA,
          pltpu.SemaphoreType.DMA,
      ],
  )
  def gather_kernel(hcat_hbm, traj_hbm, seq_out,
                    idx_v, buf0, buf1, g0, g1, w0, w1):
    cid = lax.axis_index("c")
    sid = lax.axis_index("s")
    wid = cid * NS + sid
    pltpu.sync_copy(traj_hbm.at[wid], idx_v)
    base = wid * GCPT

    bufs = (buf0, buf1)
    gsems = (g0, g1)
    wsems = (w0, w1)

    def out_rows(j):
      return pl.ds((base + j) * GK, GK)

    pltpu.async_copy(hcat_hbm.at[idx_v.at[0]], buf0, g0)

    def body(i, carry):
      jj = i * 2
      for b in range(2):
        j = jj + b
        nj = j + 1

        @pl.when(nj < GCPT)
        def _():
          @pl.when(nj >= 2)
          def _():
            pltpu.make_async_copy(bufs[1 - b], seq_out.at[out_rows(nj - 2)],
                                  wsems[1 - b]).wait()
          pltpu.async_copy(hcat_hbm.at[idx_v.at[nj]], bufs[1 - b],
                           gsems[1 - b])

        pltpu.make_async_copy(hcat_hbm.at[idx_v.at[j]], bufs[b],
                              gsems[b]).wait()
        pltpu.async_copy(bufs[b], seq_out.at[out_rows(j)], wsems[b])
      return carry

    lax.fori_loop(0, GCPT // 2, body, 0)
    pltpu.make_async_copy(buf0, seq_out.at[out_rows(GCPT - 2)], w0).wait()
    pltpu.make_async_copy(buf1, seq_out.at[out_rows(GCPT - 1)], w1).wait()

  return gather_kernel(hcat, traj3d)


def _tc_init(x0, Wenc, benc):
  def body(x_ref, w_ref, b_ref, h0_ref, az_ref, ar_ref, ah_ref):
    a = jnp.dot(x_ref[...], w_ref[...], preferred_element_type=F32) + b_ref[...]
    h0_ref[...] = jnp.tanh(a[:, :H])
    az_ref[...] = a[:, H:2 * H]
    ar_ref[...] = a[:, 2 * H:3 * H]
    ah_ref[...] = a[:, 3 * H:]

  nb = N // RB
  row_spec = pl.BlockSpec((RB, H), lambda i: (i, 0))
  out = jax.ShapeDtypeStruct((N, H), F32)
  return pl.pallas_call(
      body,
      grid=(nb,),
      in_specs=[pl.BlockSpec((RB, D), lambda i: (i, 0)),
                pl.BlockSpec((D, 4 * H), lambda i: (0, 0)),
                pl.BlockSpec((1, 4 * H), lambda i: (0, 0))],
      out_specs=[row_spec] * 4,
      out_shape=[out] * 4,
  )(x0, Wenc, benc)


def _tc_cell(p0, p1, Axz, Axr, Axh, Wzr, WhhT, id128=None):
  last = id128 is not None

  def body(*refs):
    if last:
      (p0_ref, p1_ref, az_ref, ar_ref, ah_ref,
       wzr_ref, wh_ref, id_ref, h_ref) = refs
    else:
      (p0_ref, p1_ref, az_ref, ar_ref, ah_ref,
       wzr_ref, wh_ref, h_ref) = refs
    hag = p0_ref[...] + p1_ref[...]
    zr = jnp.dot(hag, wzr_ref[...], preferred_element_type=F32)
    z = jax.nn.sigmoid(az_ref[...] + zr[:, :H])
    r = jax.nn.sigmoid(ar_ref[...] + zr[:, H:])
    ht = jnp.tanh(
        ah_ref[...] + jnp.dot(r * hag, wh_ref[...], preferred_element_type=F32))
    h = (1.0 - z) * hag + z * ht
    if last:
      i = pl.program_id(0)
      rowid = i * RB + lax.broadcasted_iota(jnp.int32, (RB, 1), 0)
      idz = jnp.where(rowid == 0, 0.0, id_ref[...])
      h_ref[:, :H] = h
      h_ref[:, H:H + DID] = idz
      h_ref[:, H + DID:] = jnp.zeros((RB, H - DID), F32)
    else:
      h_ref[...] = h

  nb = N // RB
  row_spec = pl.BlockSpec((RB, H), lambda i: (i, 0))
  in_specs = [row_spec] * 5 + [pl.BlockSpec((H, 2 * H), lambda i: (0, 0)),
                               pl.BlockSpec((H, H), lambda i: (0, 0))]
  args = [p0, p1, Axz, Axr, Axh, Wzr, WhhT]
  if last:
    in_specs.append(pl.BlockSpec((RB, DID), lambda i: (i, 0)))
    args.append(id128)
    out_spec = pl.BlockSpec((RB, 2 * H), lambda i: (i, 0))
    out_shape = jax.ShapeDtypeStruct((N, 2 * H), F32)
  else:
    out_spec = row_spec
    out_shape = jax.ShapeDtypeStruct((N, H), F32)
  return pl.pallas_call(
      body,
      grid=(nb,),
      in_specs=in_specs,
      out_specs=out_spec,
      out_shape=out_shape,
  )(*args)


def _tc_birnn(seq, lens2d, Wcat, bcat, Wbd, bhhcat):
  def body(xf_ref, xb_ref, len_ref, wc_ref, bc_ref, w_ref, bhh_ref,
           hf_ref, hb_ref):
    i = pl.program_id(0)

    @pl.when(i == 0)
    def _():
      hf_ref[...] = jnp.zeros((B, H), F32)
      hb_ref[...] = jnp.zeros((B, H), F32)

    lens = jnp.clip(len_ref[...], 1, L)
    wc = wc_ref[...]
    bc = bc_ref[...]

    def gru(gi, ghd, hprev, tcur, lh):
      r = jax.nn.sigmoid(gi[:, 0:H] + ghd[:, 0:H])
      z = jax.nn.sigmoid(gi[:, H:2 * H] + ghd[:, H:2 * H])
      n = jnp.tanh(gi[:, 2 * H:] + r * ghd[:, 2 * H:])
      hnew = (1.0 - z) * n + z * hprev
      return jnp.where(tcur < lh, hnew, hprev)

    HB = B // 2
    hf0 = hf_ref[...]
    hb0 = hb_ref[...]
    hs = [hf0[:HB], hf0[HB:], hb0[:HB], hb0[HB:]]
    for s in range(TSTEP):
      t = i * TSTEP + s
      xf = xf_ref[0, s]
      xb = xb_ref[0, TSTEP - 1 - s]
      new = []
      for half in range(2):
        gif = jnp.dot(xf[half * HB:(half + 1) * HB], wc[:, :3 * H],
                      preferred_element_type=F32) + bc[:, :3 * H]
        gib = jnp.dot(xb[half * HB:(half + 1) * HB], wc[:, 3 * H:],
                      preferred_element_type=F32) + bc[:, 3 * H:]
        hfh = hs[half]
        hbh = hs[2 + half]
        x = jnp.concatenate([hfh, hbh], axis=1)
        gh = jnp.dot(x, w_ref[...], preferred_element_type=F32) + bhh_ref[...]
        lh = lens[half * HB:(half + 1) * HB]
        new.append((gru(gif, gh[:, :3 * H], hfh, t, lh),
                    gru(gib, gh[:, 3 * H:], hbh, L - 1 - t, lh)))
      hs = [new[0][0], new[1][0], new[0][1], new[1][1]]
    hf_ref[0:HB] = hs[0]
    hf_ref[HB:] = hs[1]
    hb_ref[0:HB] = hs[2]
    hb_ref[HB:] = hs[3]

  seq4 = seq.reshape(L // TSTEP, TSTEP, B, 2 * H)
  out = jax.ShapeDtypeStruct((B, H), F32)
  return pl.pallas_call(
      body,
      grid=(L // TSTEP,),
      in_specs=[pl.BlockSpec((1, TSTEP, B, 2 * H), lambda i: (i, 0, 0, 0)),
                pl.BlockSpec((1, TSTEP, B, 2 * H),
                             lambda i: (L // TSTEP - 1 - i, 0, 0, 0)),
                pl.BlockSpec((B, 1), lambda i: (0, 0)),
                pl.BlockSpec((2 * H, 6 * H), lambda i: (0, 0)),
                pl.BlockSpec((1, 6 * H), lambda i: (0, 0)),
                pl.BlockSpec((2 * H, 6 * H), lambda i: (0, 0)),
                pl.BlockSpec((1, 6 * H), lambda i: (0, 0))],
      out_specs=[pl.BlockSpec((B, H), lambda i: (0, 0))] * 2,
      out_shape=[out, out],
  )(seq4, seq4, lens2d, Wcat, bcat, Wbd, bhhcat)


def _tc_head(hf, hb, dyn, ln_g, ln_b, W1T_s, W1T_d, b1, w2, b2):
  def body(hf_ref, hb_ref, dyn_ref, g_ref, be_ref, w1s_ref, w1d_ref,
           b1_ref, w2_ref, b2_ref, out_ref):
    state = jnp.concatenate([hf_ref[...], hb_ref[...]], axis=1)
    mu = jnp.mean(state, axis=1, keepdims=True)
    var = jnp.mean(jnp.square(state - mu), axis=1, keepdims=True)
    state = (state - mu) * jax.lax.rsqrt(var + 1e-5) * g_ref[...] + be_ref[...]
    z1 = (jnp.dot(state, w1s_ref[...], preferred_element_type=F32)
          + jnp.dot(dyn_ref[...], w1d_ref[...], preferred_element_type=F32)
          + b1_ref[...])
    h1 = 0.5 * z1 * (1.0 + lax.erf(z1 * 0.7071067811865476))
    out_ref[0, :] = jnp.sum(h1 * w2_ref[...], axis=1) + b2_ref[0, 0]

  return pl.pallas_call(
      body,
      in_specs=[pl.BlockSpec((B, H), lambda: (0, 0)),
                pl.BlockSpec((B, H), lambda: (0, 0)),
                pl.BlockSpec((B, DDYN), lambda: (0, 0)),
                pl.BlockSpec((1, 2 * H), lambda: (0, 0)),
                pl.BlockSpec((1, 2 * H), lambda: (0, 0)),
                pl.BlockSpec((2 * H, H), lambda: (0, 0)),
                pl.BlockSpec((DDYN, H), lambda: (0, 0)),
                pl.BlockSpec((1, H), lambda: (0, 0)),
                pl.BlockSpec((1, H), lambda: (0, 0)),
                pl.BlockSpec((1, 1), lambda: (0, 0))],
      out_specs=pl.BlockSpec((1, B), lambda: (0, 0)),
      out_shape=jax.ShapeDtypeStruct((1, B), F32),
  )(hf, hb, dyn, ln_g, ln_b, W1T_s, W1T_d, b1, w2, b2)


def kernel(x0, edge_index, traj, lengths, dyn_feat, params):
  p = params
  Wenc = jnp.concatenate(
      [p['We'].T, p['Wz'][:, :D].T, p['Wr'][:, :D].T, p['Wh'][:, :D].T], axis=1)
  benc = jnp.concatenate(
      [p['be'], p['bz'], p['br'], p['bh']]).reshape(1, 4 * H)
  Wzr = jnp.concatenate([p['Wz'][:, D:].T, p['Wr'][:, D:].T], axis=1)
  WhhT = p['Wh'][:, D:].T

  Wcat = jnp.concatenate([
      jnp.concatenate([p['Wih_f'][:, :H].T, p['Wih_b'][:, :H].T], axis=1),
      jnp.concatenate([p['Wih_f'][:, H:].T, p['Wih_b'][:, H:].T], axis=1),
      jnp.zeros((H - DID, 6 * H), F32)], axis=0)
  bcat = jnp.concatenate([p['bih_f'], p['bih_b']]).reshape(1, 6 * H)
  Wbd = jnp.concatenate([
      jnp.concatenate([p['Whh_f'].T, jnp.zeros((H, 3 * H), F32)], axis=1),
      jnp.concatenate([jnp.zeros((H, 3 * H), F32), p['Whh_b'].T], axis=1)],
      axis=0)
  bhhcat = jnp.concatenate([p['bhh_f'], p['bhh_b']]).reshape(1, 6 * H)

  id128 = p['id_table']
  ln_g = p['ln_g'].reshape(1, 2 * H)
  ln_b = p['ln_b'].reshape(1, 2 * H)
  W1T_s = p['W1'][:, :2 * H].T
  W1T_d = p['W1'][:, 2 * H:].T
  b1 = p['b1'].reshape(1, H)
  w2 = p['W2'].reshape(1, H)
  b2 = p['b2'].reshape(1, 1)

  edges4d = edge_index.reshape(2, NW, ECPT, EK)
  zeros_tile = jnp.zeros((ROWS_PT, H), F32)
  traj3d = traj.T.reshape(NW, GCPT, GK)
  lens2d = lengths.reshape(B, 1)

  h, Axz, Axr, Axh = _tc_init(x0, Wenc, benc)
  for s in range(STEPS):
    pa, pb = _sc_scatter_add(h, edges4d, zeros_tile)
    h = _tc_cell(pa, pb, Axz, Axr, Axh, Wzr, WhhT,
                 id128=id128 if s == STEPS - 1 else None)

  seq = _sc_traj_gather(h, traj3d)
  hf, hb = _tc_birnn(seq, lens2d, Wcat, bcat, Wbd, bhhcat)
  out = _tc_head(hf, hb, dyn_feat, ln_g, ln_b, W1T_s, W1T_d, b1, w2, b2)
  return out.reshape(B)


# speedup vs baseline: 1.0026x; 1.0026x over previous
import functools

import jax
import jax.numpy as jnp
from jax import lax
from jax.experimental import pallas as pl
from jax.experimental.pallas import tpu as pltpu
from jax.experimental.pallas import tpu_sc as plsc

F32 = jnp.float32

N = 10000
E = 320000
D = 128
H = 128
DID = 32
DDYN = 16
B = 1024
L = 50
STEPS = 3

NC = 2
NS = 16
NW = NC * NS

NP = 10112
EK = 125
ECPT = E // EK // NW
IG = 8
NG = ECPT // IG
ROWS_PT = NP // NS

LB = B * L
GK = 80
GCPT = LB // GK // NW

RB = 2000
TSTEP = 5


def _sc_scatter_add(h, edges4d, zeros_tile):
  mesh = plsc.VectorSubcoreMesh(core_axis_name="c", subcore_axis_name="s",
                                num_cores=NC, num_subcores=NS)

  @functools.partial(
      pl.kernel,
      out_type=[jax.ShapeDtypeStruct((NP, H), F32),
                jax.ShapeDtypeStruct((NP, H), F32)],
      mesh=mesh,
      scratch_types=[
          pltpu.VMEM((2, IG, EK), jnp.int32),
          pltpu.VMEM((2, IG, EK), jnp.int32),
          pltpu.VMEM((EK, H), F32),
          pltpu.VMEM((EK, H), F32),
          pltpu.VMEM_SHARED((NP, H), F32),
          pltpu.SemaphoreType.DMA,
          pltpu.SemaphoreType.DMA,
          pltpu.SemaphoreType.DMA,
          pltpu.SemaphoreType.DMA,
      ],
  )
  def scatter_kernel(h_hbm, edges_hbm, z_hbm, out0, out1,
                     sidx, didx, rows0, rows1, acc_sh, gs0, gs1, is0, is1):
    cid = lax.axis_index("c")
    sid = lax.axis_index("s")
    wid = cid * NS + sid
    my_rows = pl.ds(sid * ROWS_PT, ROWS_PT)
    src_hbm = edges_hbm.at[0]
    dst_hbm = edges_hbm.at[1]
    pltpu.sync_copy(z_hbm, acc_sh.at[my_rows])
    pltpu.sync_copy(src_hbm.at[wid, pl.ds(0, IG)], sidx.at[0])
    pltpu.sync_copy(dst_hbm.at[wid, pl.ds(0, IG)], didx.at[0])
    plsc.subcore_barrier()

    bufs = (rows0, rows1)
    gsems = (gs0, gs1)
    SPLITS = ((0, 64), (64, EK - 64))

    def fire(gb_, k_, buf, sem):
      for (o, n) in SPLITS:
        pltpu.async_copy(h_hbm.at[sidx.at[gb_, k_, pl.ds(o, n)]],
                         buf.at[pl.ds(o, n)], sem)

    def drain(gb_, k_, buf, sem):
      for (o, n) in SPLITS:
        pltpu.make_async_copy(h_hbm.at[sidx.at[gb_, k_, pl.ds(o, n)]],
                              buf.at[pl.ds(o, n)], sem).wait()

    fire(0, 0, rows0, gs0)

    def group(g, carry):
      gb = g % 2
      ngb = (g + 1) % 2
      nxt = pl.ds((g + 1) * IG, IG)

      @pl.when(g + 1 < NG)
      def _():
        pltpu.async_copy(src_hbm.at[wid, nxt], sidx.at[ngb], is0)
        pltpu.async_copy(dst_hbm.at[wid, nxt], didx.at[ngb], is1)

      for k in range(IG):
        if k < IG - 1:
          fire(gb, k + 1, bufs[(k + 1) % 2], gsems[(k + 1) % 2])
        else:

          @pl.when(g + 1 < NG)
          def _():
            pltpu.make_async_copy(src_hbm.at[wid, nxt], sidx.at[ngb],
                                  is0).wait()
            pltpu.make_async_copy(dst_hbm.at[wid, nxt], didx.at[ngb],
                                  is1).wait()
            fire(ngb, 0, bufs[0], gsems[0])

        drain(gb, k, bufs[k % 2], gsems[k % 2])
        pltpu.sync_copy(bufs[k % 2], acc_sh.at[didx.at[gb, k]], add=True)
      return carry

    lax.fori_loop(0, NG, group, 0)
    plsc.subcore_barrier()

    @pl.when(cid == 0)
    def _():
      pltpu.sync_copy(acc_sh.at[my_rows], out0.at[my_rows])

    @pl.when(cid == 1)
    def _():
      pltpu.sync_copy(acc_sh.at[my_rows], out1.at[my_rows])

  return scatter_kernel(h, edges4d, zeros_tile)


def _sc_traj_gather(hcat, traj3d):
  mesh = plsc.VectorSubcoreMesh(core_axis_name="c", subcore_axis_name="s",
                                num_cores=NC, num_subcores=NS)

  @functools.partial(
      pl.kernel,
      out_type=jax.ShapeDtypeStruct((LB, 2 * H), F32),
      mesh=mesh,
      scratch_types=[
          pltpu.VMEM((GCPT, GK), jnp.int32),
          pltpu.VMEM((GK, 2 * H), F32),
          pltpu.VMEM((GK, 2 * H), F32),
          pltpu.SemaphoreType.DMA,
          pltpu.SemaphoreType.DMA,
          pltpu.SemaphoreType.DMA,
          pltpu.SemaphoreType.DMA,
      ],
  )
  def gather_kernel(hcat_hbm, traj_hbm, seq_out,
                    idx_v, buf0, buf1, g0, g1, w0, w1):
    cid = lax.axis_index("c")
    sid = lax.axis_index("s")
    wid = cid * NS + sid
    pltpu.sync_copy(traj_hbm.at[wid], idx_v)
    base = wid * GCPT

    bufs = (buf0, buf1)
    gsems = (g0, g1)
    wsems = (w0, w1)

    def out_rows(j):
      return pl.ds((base + j) * GK, GK)

    pltpu.async_copy(hcat_hbm.at[idx_v.at[0]], buf0, g0)

    def body(i, carry):
      jj = i * 2
      for b in range(2):
        j = jj + b
        nj = j + 1

        @pl.when(nj < GCPT)
        def _():
          @pl.when(nj >= 2)
          def _():
            pltpu.make_async_copy(bufs[1 - b], seq_out.at[out_rows(nj - 2)],
                                  wsems[1 - b]).wait()
          pltpu.async_copy(hcat_hbm.at[idx_v.at[nj]], bufs[1 - b],
                           gsems[1 - b])

        pltpu.make_async_copy(hcat_hbm.at[idx_v.at[j]], bufs[b],
                              gsems[b]).wait()
        pltpu.async_copy(bufs[b], seq_out.at[out_rows(j)], wsems[b])
      return carry

    lax.fori_loop(0, GCPT // 2, body, 0)
    pltpu.make_async_copy(buf0, seq_out.at[out_rows(GCPT - 2)], w0).wait()
    pltpu.make_async_copy(buf1, seq_out.at[out_rows(GCPT - 1)], w1).wait()

  return gather_kernel(hcat, traj3d)


def _tc_init(x0, Wenc, benc):
  def body(x_ref, w_ref, b_ref, h0_ref, az_ref, ar_ref, ah_ref):
    a = jnp.dot(x_ref[...], w_ref[...], preferred_element_type=F32) + b_ref[...]
    h0_ref[...] = jnp.tanh(a[:, :H])
    az_ref[...] = a[:, H:2 * H]
    ar_ref[...] = a[:, 2 * H:3 * H]
    ah_ref[...] = a[:, 3 * H:]

  nb = N // RB
  row_spec = pl.BlockSpec((RB, H), lambda i: (i, 0))
  out = jax.ShapeDtypeStruct((N, H), F32)
  return pl.pallas_call(
      body,
      grid=(nb,),
      in_specs=[pl.BlockSpec((RB, D), lambda i: (i, 0)),
                pl.BlockSpec((D, 4 * H), lambda i: (0, 0)),
                pl.BlockSpec((1, 4 * H), lambda i: (0, 0))],
      out_specs=[row_spec] * 4,
      out_shape=[out] * 4,
  )(x0, Wenc, benc)


def _tc_cell(p0, p1, Axz, Axr, Axh, Wzr, WhhT, id128=None):
  last = id128 is not None

  def body(*refs):
    if last:
      (p0_ref, p1_ref, az_ref, ar_ref, ah_ref,
       wzr_ref, wh_ref, id_ref, h_ref) = refs
    else:
      (p0_ref, p1_ref, az_ref, ar_ref, ah_ref,
       wzr_ref, wh_ref, h_ref) = refs
    hag = p0_ref[...] + p1_ref[...]
    zr = jnp.dot(hag, wzr_ref[...], preferred_element_type=F32)
    z = jax.nn.sigmoid(az_ref[...] + zr[:, :H])
    r = jax.nn.sigmoid(ar_ref[...] + zr[:, H:])
    ht = jnp.tanh(
        ah_ref[...] + jnp.dot(r * hag, wh_ref[...], preferred_element_type=F32))
    h = (1.0 - z) * hag + z * ht
    if last:
      i = pl.program_id(0)
      rowid = i * RB + lax.broadcasted_iota(jnp.int32, (RB, 1), 0)
      idz = jnp.where(rowid == 0, 0.0, id_ref[...])
      h_ref[:, :H] = h
      h_ref[:, H:H + DID] = idz
      h_ref[:, H + DID:] = jnp.zeros((RB, H - DID), F32)
    else:
      h_ref[...] = h

  nb = N // RB
  row_spec = pl.BlockSpec((RB, H), lambda i: (i, 0))
  in_specs = [row_spec] * 5 + [pl.BlockSpec((H, 2 * H), lambda i: (0, 0)),
                               pl.BlockSpec((H, H), lambda i: (0, 0))]
  args = [p0, p1, Axz, Axr, Axh, Wzr, WhhT]
  if last:
    in_specs.append(pl.BlockSpec((RB, DID), lambda i: (i, 0)))
    args.append(id128)
    out_spec = pl.BlockSpec((RB, 2 * H), lambda i: (i, 0))
    out_shape = jax.ShapeDtypeStruct((N, 2 * H), F32)
  else:
    out_spec = row_spec
    out_shape = jax.ShapeDtypeStruct((N, H), F32)
  return pl.pallas_call(
      body,
      grid=(nb,),
      in_specs=in_specs,
      out_specs=out_spec,
      out_shape=out_shape,
  )(*args)


def _tc_birnn(seq, lens2d, Wcat, bcat, Wbd, bhhcat):
  def body(xf_ref, xb_ref, len_ref, wc_ref, bc_ref, w_ref, bhh_ref,
           hf_ref, hb_ref):
    i = pl.program_id(0)

    @pl.when(i == 0)
    def _():
      hf_ref[...] = jnp.zeros((B, H), F32)
      hb_ref[...] = jnp.zeros((B, H), F32)

    lens = jnp.clip(len_ref[...], 1, L)
    wc = wc_ref[...]
    bc = bc_ref[...]

    def gru(gi, ghd, hprev, tcur, lh):
      r = jax.nn.sigmoid(gi[:, 0:H] + ghd[:, 0:H])
      z = jax.nn.sigmoid(gi[:, H:2 * H] + ghd[:, H:2 * H])
      n = jnp.tanh(gi[:, 2 * H:] + r * ghd[:, 2 * H:])
      hnew = (1.0 - z) * n + z * hprev
      return jnp.where(tcur < lh, hnew, hprev)

    HB = B // 2
    hf0 = hf_ref[...]
    hb0 = hb_ref[...]
    hs = [hf0[:HB], hf0[HB:], hb0[:HB], hb0[HB:]]
    for s in range(TSTEP):
      t = i * TSTEP + s
      xf = xf_ref[0, s]
      xb = xb_ref[0, TSTEP - 1 - s]
      new = []
      for half in range(2):
        gif = jnp.dot(xf[half * HB:(half + 1) * HB], wc[:, :3 * H],
                      preferred_element_type=F32) + bc[:, :3 * H]
        gib = jnp.dot(xb[half * HB:(half + 1) * HB], wc[:, 3 * H:],
                      preferred_element_type=F32) + bc[:, 3 * H:]
        hfh = hs[half]
        hbh = hs[2 + half]
        x = jnp.concatenate([hfh, hbh], axis=1)
        gh = jnp.dot(x, w_ref[...], preferred_element_type=F32) + bhh_ref[...]
        lh = lens[half * HB:(half + 1) * HB]
        new.append((gru(gif, gh[:, :3 * H], hfh, t, lh),
                    gru(gib, gh[:, 3 * H:], hbh, L - 1 - t, lh)))
      hs = [new[0][0], new[1][0], new[0][1], new[1][1]]
    hf_ref[0:HB] = hs[0]
    hf_ref[HB:] = hs[1]
    hb_ref[0:HB] = hs[2]
    hb_ref[HB:] = hs[3]

  seq4 = seq.reshape(L // TSTEP, TSTEP, B, 2 * H)
  out = jax.ShapeDtypeStruct((B, H), F32)
  return pl.pallas_call(
      body,
      grid=(L // TSTEP,),
      in_specs=[pl.BlockSpec((1, TSTEP, B, 2 * H), lambda i: (i, 0, 0, 0)),
                pl.BlockSpec((1, TSTEP, B, 2 * H),
                             lambda i: (L // TSTEP - 1 - i, 0, 0, 0)),
                pl.BlockSpec((B, 1), lambda i: (0, 0)),
                pl.BlockSpec((2 * H, 6 * H), lambda i: (0, 0)),
                pl.BlockSpec((1, 6 * H), lambda i: (0, 0)),
                pl.BlockSpec((2 * H, 6 * H), lambda i: (0, 0)),
                pl.BlockSpec((1, 6 * H), lambda i: (0, 0))],
      out_specs=[pl.BlockSpec((B, H), lambda i: (0, 0))] * 2,
      out_shape=[out, out],
  )(seq4, seq4, lens2d, Wcat, bcat, Wbd, bhhcat)


def _tc_head(hf, hb, dyn, ln_g, ln_b, W1T_s, W1T_d, b1, w2, b2):
  def body(hf_ref, hb_ref, dyn_ref, g_ref, be_ref, w1s_ref, w1d_ref,
           b1_ref, w2_ref, b2_ref, out_ref):
    state = jnp.concatenate([hf_ref[...], hb_ref[...]], axis=1)
    mu = jnp.mean(state, axis=1, keepdims=True)
    var = jnp.mean(jnp.square(state - mu), axis=1, keepdims=True)
    state = (state - mu) * jax.lax.rsqrt(var + 1e-5) * g_ref[...] + be_ref[...]
    z1 = (jnp.dot(state, w1s_ref[...], preferred_element_type=F32)
          + jnp.dot(dyn_ref[...], w1d_ref[...], preferred_element_type=F32)
          + b1_ref[...])
    h1 = 0.5 * z1 * (1.0 + lax.erf(z1 * 0.7071067811865476))
    out_ref[0, :] = jnp.sum(h1 * w2_ref[...], axis=1) + b2_ref[0, 0]

  return pl.pallas_call(
      body,
      in_specs=[pl.BlockSpec((B, H), lambda: (0, 0)),
                pl.BlockSpec((B, H), lambda: (0, 0)),
                pl.BlockSpec((B, DDYN), lambda: (0, 0)),
                pl.BlockSpec((1, 2 * H), lambda: (0, 0)),
                pl.BlockSpec((1, 2 * H), lambda: (0, 0)),
                pl.BlockSpec((2 * H, H), lambda: (0, 0)),
                pl.BlockSpec((DDYN, H), lambda: (0, 0)),
                pl.BlockSpec((1, H), lambda: (0, 0)),
                pl.BlockSpec((1, H), lambda: (0, 0)),
                pl.BlockSpec((1, 1), lambda: (0, 0))],
      out_specs=pl.BlockSpec((1, B), lambda: (0, 0)),
      out_shape=jax.ShapeDtypeStruct((1, B), F32),
  )(hf, hb, dyn, ln_g, ln_b, W1T_s, W1T_d, b1, w2, b2)


def kernel(x0, edge_index, traj, lengths, dyn_feat, params):
  p = params
  Wenc = jnp.concatenate(
      [p['We'].T, p['Wz'][:, :D].T, p['Wr'][:, :D].T, p['Wh'][:, :D].T], axis=1)
  benc = jnp.concatenate(
      [p['be'], p['bz'], p['br'], p['bh']]).reshape(1, 4 * H)
  Wzr = jnp.concatenate([p['Wz'][:, D:].T, p['Wr'][:, D:].T], axis=1)
  WhhT = p['Wh'][:, D:].T

  Wcat = jnp.concatenate([
      jnp.concatenate([p['Wih_f'][:, :H].T, p['Wih_b'][:, :H].T], axis=1),
      jnp.concatenate([p['Wih_f'][:, H:].T, p['Wih_b'][:, H:].T], axis=1),
      jnp.zeros((H - DID, 6 * H), F32)], axis=0)
  bcat = jnp.concatenate([p['bih_f'], p['bih_b']]).reshape(1, 6 * H)
  Wbd = jnp.concatenate([
      jnp.concatenate([p['Whh_f'].T, jnp.zeros((H, 3 * H), F32)], axis=1),
      jnp.concatenate([jnp.zeros((H, 3 * H), F32), p['Whh_b'].T], axis=1)],
      axis=0)
  bhhcat = jnp.concatenate([p['bhh_f'], p['bhh_b']]).reshape(1, 6 * H)

  id128 = p['id_table']
  ln_g = p['ln_g'].reshape(1, 2 * H)
  ln_b = p['ln_b'].reshape(1, 2 * H)
  W1T_s = p['W1'][:, :2 * H].T
  W1T_d = p['W1'][:, 2 * H:].T
  b1 = p['b1'].reshape(1, H)
  w2 = p['W2'].reshape(1, H)
  b2 = p['b2'].reshape(1, 1)

  edges4d = edge_index.reshape(2, NW, ECPT, EK)
  zeros_tile = jnp.zeros((ROWS_PT, H), F32)
  traj3d = traj.T.reshape(NW, GCPT, GK)
  lens2d = lengths.reshape(B, 1)

  h, Axz, Axr, Axh = _tc_init(x0, Wenc, benc)
  for s in range(STEPS):
    pa, pb = _sc_scatter_add(h, edges4d, zeros_tile)
    h = _tc_cell(pa, pb, Axz, Axr, Axh, Wzr, WhhT,
                 id128=id128 if s == STEPS - 1 else None)

  seq = _sc_traj_gather(h, traj3d)
  hf, hb = _tc_birnn(seq, lens2d, Wcat, bcat, Wbd, bhhcat)
  out = _tc_head(hf, hb, dyn_feat, ln_g, ln_b, W1T_s, W1T_d, b1, w2, b2)
  return out.reshape(B)
